# baseline (speedup 1.0000x reference)
import functools

import jax
import jax.numpy as jnp
from jax import lax
from jax.experimental import pallas as pl
from jax.experimental.pallas import tpu as pltpu
from jax.experimental.pallas import tpu_sc as plsc

N = 10000
E = 320000
D = 128
D_EDGE = 16
D_HID = 256

NC = 2
NS = 16
EPW = E // (NC * NS)
CHUNK = 80
NCHUNK = EPW // CHUNK
NPAIR = (NCHUNK - 1) // 2
NP = 10240
RPS = NP // NS


def _enc_body(attr_ref, we_ref, be_ref, out_ref):
    acc = (
        jnp.dot(attr_ref[...], we_ref[...], preferred_element_type=jnp.float32)
        + be_ref[...]
    )
    blk2 = acc.shape[0] // 2
    acc3 = acc.reshape(blk2, 2, D)
    ev = lax.bitcast_convert_type(acc3[:, 0, :].astype(jnp.bfloat16),
                                  jnp.uint16).astype(jnp.uint32)
    od = lax.bitcast_convert_type(acc3[:, 1, :].astype(jnp.bfloat16),
                                  jnp.uint16).astype(jnp.uint32)
    out_ref[...] = lax.bitcast_convert_type(ev | (od << 16), jnp.int32)


def _edge_encoder(edge_attr, W_e, b_e):
    BLK = 3200
    return pl.pallas_call(
        _enc_body,
        grid=(E // BLK,),
        in_specs=[
            pl.BlockSpec((BLK, D_EDGE), lambda i: (i, 0)),
            pl.BlockSpec((D_EDGE, D), lambda i: (0, 0)),
            pl.BlockSpec((1, D), lambda i: (0, 0)),
        ],
        out_specs=pl.BlockSpec((BLK // 2, D), lambda i: (i, 0)),
        out_shape=jax.ShapeDtypeStruct((E // 2, D), jnp.int32),
    )(edge_attr, W_e, b_e.reshape(1, D))


def _sc_body(x_hbm, src_hbm, dst_hbm, emb_hbm, out_hbm,
             src_v, dst_v, rows_v, emb_v, lsem0, lsem1, ssem0, ssem1,
             aggr_sh):
    lsem = (lsem0, lsem1)
    ssem = (ssem0, ssem1)
    c = lax.axis_index("c")
    s = lax.axis_index("s")

    def zfill(i, carry):
        for j in range(D // 16):
            rows_v[0, i, pl.ds(j * 16, 16)] = jnp.zeros((16,), jnp.float32)
        return carry

    lax.fori_loop(0, CHUNK, zfill, 0)
    for k in range(RPS // CHUNK):
        pltpu.sync_copy(rows_v.at[0],
                        aggr_sh.at[pl.ds(s * RPS + k * CHUNK, CHUNK)])
    plsc.subcore_barrier()

    ebase = (c * NS + s) * EPW

    ebase2 = (c * NS + s) * (EPW // 2)

    def fill(i, b):
        base = ebase + i * CHUNK
        base2 = ebase2 + i * (CHUNK // 2)
        pltpu.sync_copy(src_hbm.at[pl.ds(base, CHUNK)], src_v.at[b])
        pltpu.sync_copy(dst_hbm.at[pl.ds(base, CHUNK)], dst_v.at[b])
        pltpu.async_copy(emb_hbm.at[pl.ds(base2, CHUNK // 2)],
                         emb_v.at[b], lsem[b])
        pltpu.async_copy(x_hbm.at[src_v.at[b]], rows_v.at[b], lsem[b])

    def drain_loads(b):
        pltpu.make_async_copy(emb_hbm.at[pl.ds(0, CHUNK // 2)], emb_v.at[b],
                              lsem[b]).wait()
        pltpu.make_async_copy(x_hbm.at[pl.ds(0, CHUNK)], rows_v.at[b],
                              lsem[b]).wait()

    def compute(b):
        fmt = plsc.PackFormat.INTERLEAVED

        def msg_pair(p, carry):
            r0 = 2 * p
            for k in range(D // 16):
                sl = pl.ds(16 * k, 16)
                bfv = plsc.bitcast(emb_v[b, p, sl], jnp.bfloat16)
                elo, ehi = plsc.unpack(bfv, format=fmt)
                rows_v[b, r0, sl] = jnp.maximum(rows_v[b, r0, sl] + elo, 0.0)
                rows_v[b, r0 + 1, sl] = jnp.maximum(
                    rows_v[b, r0 + 1, sl] + ehi, 0.0)
            return carry

        lax.fori_loop(0, CHUNK // 2, msg_pair, 0)

    def issue_scatter(b):
        pltpu.async_copy(rows_v.at[b], aggr_sh.at[dst_v.at[b]], ssem[b],
                         add=True)

    def drain_scatter(b):
        pltpu.make_async_copy(rows_v.at[b], aggr_sh.at[dst_v.at[b]],
                              ssem[b]).wait()

    fill(0, 0)
    fill(1, 1)

    def pair_body(g, carry):
        for b in range(2):
            drain_loads(b)
            compute(b)
            issue_scatter(b)

        @pl.when(g < NPAIR - 1)
        def _():
            for b in range(2):
                drain_scatter(b)
                fill(2 * g + 2 + b, b)

        return carry

    lax.fori_loop(0, NPAIR, pair_body, 0)

    drain_scatter(0)
    fill(NCHUNK - 1, 0)
    drain_loads(0)
    compute(0)
    issue_scatter(0)
    drain_scatter(1)
    drain_scatter(0)

    plsc.subcore_barrier()
    pltpu.sync_copy(aggr_sh.at[pl.ds(s * RPS, RPS)], out_hbm.at[c * NS + s])


def _sc_aggregate(x, src, dst, emb_pairs):
    mesh = plsc.VectorSubcoreMesh(core_axis_name="c", subcore_axis_name="s")
    f = pl.kernel(
        _sc_body,
        out_type=jax.ShapeDtypeStruct((NC * NS, RPS, D), jnp.float32),
        mesh=mesh,
        compiler_params=pltpu.CompilerParams(needs_layout_passes=False),
        scratch_types=[
            pltpu.VMEM((2, CHUNK), jnp.int32),
            pltpu.VMEM((2, CHUNK), jnp.int32),
            pltpu.VMEM((2, CHUNK, D), jnp.float32),
            pltpu.VMEM((2, CHUNK // 2, D), jnp.int32),
            pltpu.SemaphoreType.DMA,
            pltpu.SemaphoreType.DMA,
            pltpu.SemaphoreType.DMA,
            pltpu.SemaphoreType.DMA,
            pltpu.VMEM_SHARED((NP, D), jnp.float32),
        ],
    )
    return f(x, src, dst, emb_pairs)


def _mlp_body(x_ref, a0_ref, a1_ref, epsv_ref, w1_ref, b1_ref, w2_ref, b2_ref,
              out_ref):
    h = epsv_ref[...] * x_ref[...] + a0_ref[...] + a1_ref[...]
    h = jnp.dot(h, w1_ref[...], preferred_element_type=jnp.float32) + b1_ref[...]
    h = jnp.maximum(h, 0.0)
    out_ref[...] = (
        jnp.dot(h, w2_ref[...], preferred_element_type=jnp.float32) + b2_ref[...]
    )


def _mlp(x, a0, a1, epsv, W1f, b1f, W2f, b2f):
    BLK = 1000
    return pl.pallas_call(
        _mlp_body,
        grid=(N // BLK,),
        in_specs=[
            pl.BlockSpec((BLK, D), lambda i: (i, 0)),
            pl.BlockSpec((BLK, D), lambda i: (i, 0)),
            pl.BlockSpec((BLK, D), lambda i: (i, 0)),
            pl.BlockSpec((1, D), lambda i: (0, 0)),
            pl.BlockSpec((D, D_HID), lambda i: (0, 0)),
            pl.BlockSpec((1, D_HID), lambda i: (0, 0)),
            pl.BlockSpec((D_HID, D), lambda i: (0, 0)),
            pl.BlockSpec((1, D), lambda i: (0, 0)),
        ],
        out_specs=pl.BlockSpec((BLK, D), lambda i: (i, 0)),
        out_shape=jax.ShapeDtypeStruct((N, D), jnp.float32),
    )(x, a0, a1, epsv, W1f, b1f, W2f, b2f)


def kernel(input_feature, edge_index, edge_attr, W_e, b_e, eps, W1, b1,
           gamma1, beta1, mean1, var1, W2, b2, gamma2, beta2, mean2, var2):
    src = edge_index[0]
    dst = edge_index[1]

    emb_pairs = _edge_encoder(edge_attr, W_e, b_e)
    partials = _sc_aggregate(input_feature, src, dst, emb_pairs)
    partials = partials.reshape(NC, NP, D)
    a0 = partials[0]
    a1 = partials[1]

    scale1 = gamma1 / jnp.sqrt(var1 + 1e-5)
    W1f = W1 * scale1[None, :]
    b1f = ((b1 - mean1) * scale1 + beta1).reshape(1, D_HID)
    scale2 = gamma2 / jnp.sqrt(var2 + 1e-5)
    W2f = W2 * scale2[None, :]
    b2f = ((b2 - mean2) * scale2 + beta2).reshape(1, D)
    epsv = jnp.full((1, D), 1.0 + eps, dtype=jnp.float32)

    return _mlp(input_feature, a0, a1, epsv, W1f, b1f, W2f, b2f)


# speedup vs baseline: 1.1467x; 1.1467x over previous
import functools

import jax
import jax.numpy as jnp
from jax import lax
from jax.experimental import pallas as pl
from jax.experimental.pallas import tpu as pltpu
from jax.experimental.pallas import tpu_sc as plsc

N = 10000
E = 320000
D = 128
D_EDGE = 16
D_HID = 256

NC = 2
NS = 16
EPW = E // (NC * NS)
CHUNK = 80
NCHUNK = EPW // CHUNK
NPAIR = (NCHUNK - 1) // 2
NP = 10240
RPS = NP // NS


def _enc_body(attr_ref, we_ref, be_ref, out_ref):
    acc = (
        jnp.dot(attr_ref[...], we_ref[...], preferred_element_type=jnp.float32)
        + be_ref[...]
    )
    out_ref[...] = pltpu.bitcast(acc.astype(jnp.bfloat16), jnp.int32)


def _edge_encoder(edge_attr, W_e, b_e):
    BLK = 3200
    return pl.pallas_call(
        _enc_body,
        grid=(E // BLK,),
        in_specs=[
            pl.BlockSpec((BLK, D_EDGE), lambda i: (i, 0)),
            pl.BlockSpec((D_EDGE, D), lambda i: (0, 0)),
            pl.BlockSpec((1, D), lambda i: (0, 0)),
        ],
        out_specs=pl.BlockSpec((BLK // 2, D), lambda i: (i, 0)),
        out_shape=jax.ShapeDtypeStruct((E // 2, D), jnp.int32),
    )(edge_attr, W_e, b_e.reshape(1, D))


def _sc_body(x_hbm, src_hbm, dst_hbm, emb_hbm, out_hbm,
             src_v, dst_v, rows_v, emb_v, lsem0, lsem1, ssem0, ssem1,
             aggr_sh):
    lsem = (lsem0, lsem1)
    ssem = (ssem0, ssem1)
    c = lax.axis_index("c")
    s = lax.axis_index("s")

    def zfill(i, carry):
        for j in range(D // 16):
            rows_v[0, i, pl.ds(j * 16, 16)] = jnp.zeros((16,), jnp.float32)
        return carry

    lax.fori_loop(0, CHUNK, zfill, 0)
    for k in range(RPS // CHUNK):
        pltpu.sync_copy(rows_v.at[0],
                        aggr_sh.at[pl.ds(s * RPS + k * CHUNK, CHUNK)])
    plsc.subcore_barrier()

    ebase = (c * NS + s) * EPW
    ebase2 = (c * NS + s) * (EPW // 2)

    def fill(i, b):
        base = ebase + i * CHUNK
        base2 = ebase2 + i * (CHUNK // 2)
        pltpu.sync_copy(src_hbm.at[pl.ds(base, CHUNK)], src_v.at[b])
        pltpu.sync_copy(dst_hbm.at[pl.ds(base, CHUNK)], dst_v.at[b])
        pltpu.async_copy(emb_hbm.at[pl.ds(base2, CHUNK // 2)], emb_v.at[b],
                         lsem[b])
        pltpu.async_copy(x_hbm.at[src_v.at[b]], rows_v.at[b], lsem[b])

    def drain_loads(b):
        pltpu.make_async_copy(emb_hbm.at[pl.ds(0, CHUNK // 2)], emb_v.at[b],
                              lsem[b]).wait()
        pltpu.make_async_copy(x_hbm.at[pl.ds(0, CHUNK)], rows_v.at[b],
                              lsem[b]).wait()

    HI_MASK = jnp.int32(-65536)

    def compute(b):
        def msg_pair(p, carry):
            r0 = 2 * p
            for j in range(D // 16):
                sl = pl.ds(j * 16, 16)
                w = emb_v[b, p, sl]
                elo = plsc.bitcast(jnp.left_shift(w, 16), jnp.float32)
                ehi = plsc.bitcast(jnp.bitwise_and(w, HI_MASK), jnp.float32)
                rows_v[b, r0, sl] = jnp.maximum(rows_v[b, r0, sl] + elo, 0.0)
                rows_v[b, r0 + 1, sl] = jnp.maximum(
                    rows_v[b, r0 + 1, sl] + ehi, 0.0)
            return carry

        lax.fori_loop(0, CHUNK // 2, msg_pair, 0)

    def issue_scatter(b):
        pltpu.async_copy(rows_v.at[b], aggr_sh.at[dst_v.at[b]], ssem[b],
                         add=True)

    def drain_scatter(b):
        pltpu.make_async_copy(rows_v.at[b], aggr_sh.at[dst_v.at[b]],
                              ssem[b]).wait()

    fill(0, 0)
    fill(1, 1)

    def pair_body(g, carry):
        for b in range(2):
            drain_loads(b)
            compute(b)
            issue_scatter(b)

        @pl.when(g < NPAIR - 1)
        def _():
            for b in range(2):
                drain_scatter(b)
                fill(2 * g + 2 + b, b)

        return carry

    lax.fori_loop(0, NPAIR, pair_body, 0)

    drain_scatter(0)
    fill(NCHUNK - 1, 0)
    drain_loads(0)
    compute(0)
    issue_scatter(0)
    drain_scatter(1)
    drain_scatter(0)

    plsc.subcore_barrier()
    pltpu.sync_copy(aggr_sh.at[pl.ds(s * RPS, RPS)], out_hbm.at[c * NS + s])


def _sc_aggregate(x, src, dst, emb):
    mesh = plsc.VectorSubcoreMesh(core_axis_name="c", subcore_axis_name="s")
    f = pl.kernel(
        _sc_body,
        out_type=jax.ShapeDtypeStruct((NC * NS, RPS, D), jnp.float32),
        mesh=mesh,
        compiler_params=pltpu.CompilerParams(needs_layout_passes=False),
        scratch_types=[
            pltpu.VMEM((2, CHUNK), jnp.int32),
            pltpu.VMEM((2, CHUNK), jnp.int32),
            pltpu.VMEM((2, CHUNK, D), jnp.float32),
            pltpu.VMEM((2, CHUNK // 2, D), jnp.int32),
            pltpu.SemaphoreType.DMA,
            pltpu.SemaphoreType.DMA,
            pltpu.SemaphoreType.DMA,
            pltpu.SemaphoreType.DMA,
            pltpu.VMEM_SHARED((NP, D), jnp.float32),
        ],
    )
    return f(x, src, dst, emb)


def _mlp_body(x_ref, a0_ref, a1_ref, epsv_ref, w1_ref, b1_ref, w2_ref, b2_ref,
              out_ref):
    h = epsv_ref[...] * x_ref[...] + a0_ref[...] + a1_ref[...]
    h = jnp.dot(h, w1_ref[...], preferred_element_type=jnp.float32) + b1_ref[...]
    h = jnp.maximum(h, 0.0)
    out_ref[...] = (
        jnp.dot(h, w2_ref[...], preferred_element_type=jnp.float32) + b2_ref[...]
    )


def _mlp(x, a0, a1, epsv, W1f, b1f, W2f, b2f):
    BLK = 1000
    return pl.pallas_call(
        _mlp_body,
        grid=(N // BLK,),
        in_specs=[
            pl.BlockSpec((BLK, D), lambda i: (i, 0)),
            pl.BlockSpec((BLK, D), lambda i: (i, 0)),
            pl.BlockSpec((BLK, D), lambda i: (i, 0)),
            pl.BlockSpec((1, D), lambda i: (0, 0)),
            pl.BlockSpec((D, D_HID), lambda i: (0, 0)),
            pl.BlockSpec((1, D_HID), lambda i: (0, 0)),
            pl.BlockSpec((D_HID, D), lambda i: (0, 0)),
            pl.BlockSpec((1, D), lambda i: (0, 0)),
        ],
        out_specs=pl.BlockSpec((BLK, D), lambda i: (i, 0)),
        out_shape=jax.ShapeDtypeStruct((N, D), jnp.float32),
    )(x, a0, a1, epsv, W1f, b1f, W2f, b2f)


def kernel(input_feature, edge_index, edge_attr, W_e, b_e, eps, W1, b1,
           gamma1, beta1, mean1, var1, W2, b2, gamma2, beta2, mean2, var2):
    src = edge_index[0]
    dst = edge_index[1]

    emb = _edge_encoder(edge_attr, W_e, b_e)
    partials = _sc_aggregate(input_feature, src, dst, emb)
    partials = partials.reshape(NC, NP, D)
    a0 = partials[0]
    a1 = partials[1]

    scale1 = gamma1 / jnp.sqrt(var1 + 1e-5)
    W1f = W1 * scale1[None, :]
    b1f = ((b1 - mean1) * scale1 + beta1).reshape(1, D_HID)
    scale2 = gamma2 / jnp.sqrt(var2 + 1e-5)
    W2f = W2 * scale2[None, :]
    b2f = ((b2 - mean2) * scale2 + beta2).reshape(1, D)
    epsv = jnp.full((1, D), 1.0 + eps, dtype=jnp.float32)

    return _mlp(input_feature, a0, a1, epsv, W1f, b1f, W2f, b2f)


# speedup vs baseline: 1.1467x; 1.0000x over previous
import functools

import jax
import jax.numpy as jnp
from jax import lax
from jax.experimental import pallas as pl
from jax.experimental.pallas import tpu as pltpu
from jax.experimental.pallas import tpu_sc as plsc

N = 10000
E = 320000
D = 128
D_EDGE = 16
D_HID = 256

NC = 2
NS = 16
EPW = E // (NC * NS)
CHUNK = 80
NCHUNK = EPW // CHUNK
NPAIR = (NCHUNK - 1) // 2
NP = 10240
RPS = NP // NS


def _enc_body(attr_ref, we_ref, be_ref, out_ref):
    acc = (
        jnp.dot(attr_ref[...], we_ref[...], preferred_element_type=jnp.float32)
        + be_ref[...]
    )
    out_ref[...] = pltpu.bitcast(acc.astype(jnp.bfloat16), jnp.int32)


def _edge_encoder(edge_attr, W_e, b_e):
    BLK = 3200
    return pl.pallas_call(
        _enc_body,
        grid=(E // BLK,),
        in_specs=[
            pl.BlockSpec((BLK, D_EDGE), lambda i: (i, 0)),
            pl.BlockSpec((D_EDGE, D), lambda i: (0, 0)),
            pl.BlockSpec((1, D), lambda i: (0, 0)),
        ],
        out_specs=pl.BlockSpec((BLK // 2, D), lambda i: (i, 0)),
        out_shape=jax.ShapeDtypeStruct((E // 2, D), jnp.int32),
    )(edge_attr, W_e, b_e.reshape(1, D))


def _sc_body(x_hbm, src_hbm, dst_hbm, emb_hbm, out_hbm,
             src_v, dst_v, rows_v, emb_v, lsem0, lsem1, ssem0, ssem1,
             aggr_sh):
    lsem = (lsem0, lsem1)
    ssem = (ssem0, ssem1)
    c = lax.axis_index("c")
    s = lax.axis_index("s")

    def zfill(i, carry):
        for j in range(D // 16):
            rows_v[0, i, pl.ds(j * 16, 16)] = jnp.zeros((16,), jnp.float32)
        return carry

    lax.fori_loop(0, CHUNK, zfill, 0)
    for k in range(RPS // CHUNK):
        pltpu.sync_copy(rows_v.at[0],
                        aggr_sh.at[pl.ds(s * RPS + k * CHUNK, CHUNK)])
    plsc.subcore_barrier()

    ebase = (c * NS + s) * EPW
    ebase2 = (c * NS + s) * (EPW // 2)

    def fill(i, b):
        base = ebase + i * CHUNK
        base2 = ebase2 + i * (CHUNK // 2)
        pltpu.sync_copy(src_hbm.at[pl.ds(base, CHUNK)], src_v.at[b])
        pltpu.sync_copy(dst_hbm.at[pl.ds(base, CHUNK)], dst_v.at[b])
        pltpu.async_copy(emb_hbm.at[pl.ds(base2, CHUNK // 2)], emb_v.at[b],
                         lsem[b])
        pltpu.async_copy(x_hbm.at[src_v.at[b]], rows_v.at[b], lsem[b])

    def drain_loads(b):
        pltpu.make_async_copy(emb_hbm.at[pl.ds(0, CHUNK // 2)], emb_v.at[b],
                              lsem[b]).wait()
        pltpu.make_async_copy(x_hbm.at[pl.ds(0, CHUNK)], rows_v.at[b],
                              lsem[b]).wait()

    HI_MASK = jnp.int32(-65536)

    def compute(b):
        def msg_pair(p, carry):
            r0 = 2 * p
            for j in range(D // 16):
                sl = pl.ds(j * 16, 16)
                w = emb_v[b, p, sl]
                elo = lax.bitcast_convert_type(jnp.left_shift(w, 16),
                                               jnp.float32)
                ehi = lax.bitcast_convert_type(jnp.bitwise_and(w, HI_MASK),
                                               jnp.float32)
                rows_v[b, r0, sl] = jnp.maximum(rows_v[b, r0, sl] + elo, 0.0)
                rows_v[b, r0 + 1, sl] = jnp.maximum(
                    rows_v[b, r0 + 1, sl] + ehi, 0.0)
            return carry

        lax.fori_loop(0, CHUNK // 2, msg_pair, 0)

    def issue_scatter(b):
        pltpu.async_copy(rows_v.at[b], aggr_sh.at[dst_v.at[b]], ssem[b],
                         add=True)

    def drain_scatter(b):
        pltpu.make_async_copy(rows_v.at[b], aggr_sh.at[dst_v.at[b]],
                              ssem[b]).wait()

    fill(0, 0)
    fill(1, 1)

    def pair_body(g, carry):
        for b in range(2):
            drain_loads(b)
            compute(b)
            issue_scatter(b)

        @pl.when(g < NPAIR - 1)
        def _():
            for b in range(2):
                drain_scatter(b)
                fill(2 * g + 2 + b, b)

        return carry

    lax.fori_loop(0, NPAIR, pair_body, 0)

    drain_scatter(0)
    fill(NCHUNK - 1, 0)
    drain_loads(0)
    compute(0)
    issue_scatter(0)
    drain_scatter(1)
    drain_scatter(0)

    plsc.subcore_barrier()
    pltpu.sync_copy(aggr_sh.at[pl.ds(s * RPS, RPS)], out_hbm.at[c * NS + s])


def _sc_aggregate(x, src, dst, emb):
    mesh = plsc.VectorSubcoreMesh(core_axis_name="c", subcore_axis_name="s")
    f = pl.kernel(
        _sc_body,
        out_type=jax.ShapeDtypeStruct((NC * NS, RPS, D), jnp.float32),
        mesh=mesh,
        scratch_types=[
            pltpu.VMEM((2, CHUNK), jnp.int32),
            pltpu.VMEM((2, CHUNK), jnp.int32),
            pltpu.VMEM((2, CHUNK, D), jnp.float32),
            pltpu.VMEM((2, CHUNK // 2, D), jnp.int32),
            pltpu.SemaphoreType.DMA,
            pltpu.SemaphoreType.DMA,
            pltpu.SemaphoreType.DMA,
            pltpu.SemaphoreType.DMA,
            pltpu.VMEM_SHARED((NP, D), jnp.float32),
        ],
    )
    return f(x, src, dst, emb)


def _mlp_body(x_ref, a0_ref, a1_ref, epsv_ref, w1_ref, b1_ref, w2_ref, b2_ref,
              out_ref):
    h = epsv_ref[...] * x_ref[...] + a0_ref[...] + a1_ref[...]
    h = jnp.dot(h, w1_ref[...], preferred_element_type=jnp.float32) + b1_ref[...]
    h = jnp.maximum(h, 0.0)
    out_ref[...] = (
        jnp.dot(h, w2_ref[...], preferred_element_type=jnp.float32) + b2_ref[...]
    )


def _mlp(x, a0, a1, epsv, W1f, b1f, W2f, b2f):
    BLK = 1000
    return pl.pallas_call(
        _mlp_body,
        grid=(N // BLK,),
        in_specs=[
            pl.BlockSpec((BLK, D), lambda i: (i, 0)),
            pl.BlockSpec((BLK, D), lambda i: (i, 0)),
            pl.BlockSpec((BLK, D), lambda i: (i, 0)),
            pl.BlockSpec((1, D), lambda i: (0, 0)),
            pl.BlockSpec((D, D_HID), lambda i: (0, 0)),
            pl.BlockSpec((1, D_HID), lambda i: (0, 0)),
            pl.BlockSpec((D_HID, D), lambda i: (0, 0)),
            pl.BlockSpec((1, D), lambda i: (0, 0)),
        ],
        out_specs=pl.BlockSpec((BLK, D), lambda i: (i, 0)),
        out_shape=jax.ShapeDtypeStruct((N, D), jnp.float32),
    )(x, a0, a1, epsv, W1f, b1f, W2f, b2f)


def kernel(input_feature, edge_index, edge_attr, W_e, b_e, eps, W1, b1,
           gamma1, beta1, mean1, var1, W2, b2, gamma2, beta2, mean2, var2):
    src = edge_index[0]
    dst = edge_index[1]

    emb = _edge_encoder(edge_attr, W_e, b_e)
    partials = _sc_aggregate(input_feature, src, dst, emb)
    partials = partials.reshape(NC, NP, D)
    a0 = partials[0]
    a1 = partials[1]

    scale1 = gamma1 / jnp.sqrt(var1 + 1e-5)
    W1f = W1 * scale1[None, :]
    b1f = ((b1 - mean1) * scale1 + beta1).reshape(1, D_HID)
    scale2 = gamma2 / jnp.sqrt(var2 + 1e-5)
    W2f = W2 * scale2[None, :]
    b2f = ((b2 - mean2) * scale2 + beta2).reshape(1, D)
    epsv = jnp.full((1, D), 1.0 + eps, dtype=jnp.float32)

    return _mlp(input_feature, a0, a1, epsv, W1f, b1f, W2f, b2f)


# speedup vs baseline: 1.5001x; 1.3082x over previous
import functools

import jax
import jax.numpy as jnp
from jax import lax
from jax.experimental import pallas as pl
from jax.experimental.pallas import tpu as pltpu
from jax.experimental.pallas import tpu_sc as plsc

N = 10000
E = 320000
D = 128
D_EDGE = 16
D_HID = 256

NC = 2
NS = 16
EPW = E // (NC * NS)
CHUNK = 80
NCHUNK = EPW // CHUNK
NPAIR = (NCHUNK - 1) // 2
NP = 10240
RPS = NP // NS


def _enc_body(attr_ref, we_ref, be_ref, out_ref):
    acc = (
        jnp.dot(attr_ref[...], we_ref[...], preferred_element_type=jnp.float32)
        + be_ref[...]
    )
    out_ref[...] = pltpu.bitcast(acc.astype(jnp.bfloat16), jnp.int32)


def _edge_encoder(edge_attr, W_e, b_e):
    BLK = 3200
    return pl.pallas_call(
        _enc_body,
        grid=(E // BLK,),
        in_specs=[
            pl.BlockSpec((BLK, D_EDGE), lambda i: (i, 0)),
            pl.BlockSpec((D_EDGE, D), lambda i: (0, 0)),
            pl.BlockSpec((1, D), lambda i: (0, 0)),
        ],
        out_specs=pl.BlockSpec((BLK // 2, D), lambda i: (i, 0)),
        out_shape=jax.ShapeDtypeStruct((E // 2, D), jnp.int32),
    )(edge_attr, W_e, b_e.reshape(1, D))


def _sc_body(x_hbm, src_hbm, dst_hbm, emb_hbm, out_hbm,
             src_v, dst_v, rows_v, emb_v, lsem0, lsem1, ssem0, ssem1,
             aggr_sh):
    lsem = (lsem0, lsem1)
    ssem = (ssem0, ssem1)
    c = lax.axis_index("c")
    s = lax.axis_index("s")

    def zfill(i, carry):
        for j in range(D // 16):
            rows_v[0, i, pl.ds(j * 16, 16)] = jnp.zeros((16,), jnp.float32)
        return carry

    lax.fori_loop(0, CHUNK, zfill, 0)
    for k in range(RPS // CHUNK):
        pltpu.sync_copy(rows_v.at[0],
                        aggr_sh.at[pl.ds(s * RPS + k * CHUNK, CHUNK)])
    plsc.subcore_barrier()

    ebase = (c * NS + s) * EPW
    ebase2 = (c * NS + s) * (EPW // 2)

    def fill(i, b):
        base = ebase + i * CHUNK
        base2 = ebase2 + i * (CHUNK // 2)
        pltpu.sync_copy(src_hbm.at[pl.ds(base, CHUNK)], src_v.at[b])
        pltpu.sync_copy(dst_hbm.at[pl.ds(base, CHUNK)], dst_v.at[b])
        pltpu.async_copy(emb_hbm.at[pl.ds(base2, CHUNK // 2)], emb_v.at[b],
                         lsem[b])
        pltpu.async_copy(x_hbm.at[src_v.at[b]], rows_v.at[b], lsem[b])

    def drain_loads(b):
        pltpu.make_async_copy(emb_hbm.at[pl.ds(0, CHUNK // 2)], emb_v.at[b],
                              lsem[b]).wait()
        pltpu.make_async_copy(x_hbm.at[pl.ds(0, CHUNK)], rows_v.at[b],
                              lsem[b]).wait()

    HI_MASK = jnp.int32(-65536)

    def compute(b):
        @plsc.parallel_loop(0, CHUNK // 2, unroll=4)
        def msg_pair(p):
            r0 = 2 * p
            for j in range(D // 16):
                sl = pl.ds(j * 16, 16)
                w = emb_v[b, p, sl]
                elo = lax.bitcast_convert_type(jnp.left_shift(w, 16),
                                               jnp.float32)
                ehi = lax.bitcast_convert_type(jnp.bitwise_and(w, HI_MASK),
                                               jnp.float32)
                rows_v[b, r0, sl] = jnp.maximum(rows_v[b, r0, sl] + elo, 0.0)
                rows_v[b, r0 + 1, sl] = jnp.maximum(
                    rows_v[b, r0 + 1, sl] + ehi, 0.0)

    def issue_scatter(b):
        pltpu.async_copy(rows_v.at[b], aggr_sh.at[dst_v.at[b]], ssem[b],
                         add=True)

    def drain_scatter(b):
        pltpu.make_async_copy(rows_v.at[b], aggr_sh.at[dst_v.at[b]],
                              ssem[b]).wait()

    fill(0, 0)
    fill(1, 1)

    def pair_body(g, carry):
        for b in range(2):
            drain_loads(b)
            compute(b)
            issue_scatter(b)

        @pl.when(g < NPAIR - 1)
        def _():
            for b in range(2):
                drain_scatter(b)
                fill(2 * g + 2 + b, b)

        return carry

    lax.fori_loop(0, NPAIR, pair_body, 0)

    drain_scatter(0)
    fill(NCHUNK - 1, 0)
    drain_loads(0)
    compute(0)
    issue_scatter(0)
    drain_scatter(1)
    drain_scatter(0)

    plsc.subcore_barrier()
    pltpu.sync_copy(aggr_sh.at[pl.ds(s * RPS, RPS)], out_hbm.at[c * NS + s])


def _sc_aggregate(x, src, dst, emb):
    mesh = plsc.VectorSubcoreMesh(core_axis_name="c", subcore_axis_name="s")
    f = pl.kernel(
        _sc_body,
        out_type=jax.ShapeDtypeStruct((NC * NS, RPS, D), jnp.float32),
        mesh=mesh,
        scratch_types=[
            pltpu.VMEM((2, CHUNK), jnp.int32),
            pltpu.VMEM((2, CHUNK), jnp.int32),
            pltpu.VMEM((2, CHUNK, D), jnp.float32),
            pltpu.VMEM((2, CHUNK // 2, D), jnp.int32),
            pltpu.SemaphoreType.DMA,
            pltpu.SemaphoreType.DMA,
            pltpu.SemaphoreType.DMA,
            pltpu.SemaphoreType.DMA,
            pltpu.VMEM_SHARED((NP, D), jnp.float32),
        ],
    )
    return f(x, src, dst, emb)


def _mlp_body(x_ref, a0_ref, a1_ref, epsv_ref, w1_ref, b1_ref, w2_ref, b2_ref,
              out_ref):
    h = epsv_ref[...] * x_ref[...] + a0_ref[...] + a1_ref[...]
    h = jnp.dot(h, w1_ref[...], preferred_element_type=jnp.float32) + b1_ref[...]
    h = jnp.maximum(h, 0.0)
    out_ref[...] = (
        jnp.dot(h, w2_ref[...], preferred_element_type=jnp.float32) + b2_ref[...]
    )


def _mlp(x, a0, a1, epsv, W1f, b1f, W2f, b2f):
    BLK = 1000
    return pl.pallas_call(
        _mlp_body,
        grid=(N // BLK,),
        in_specs=[
            pl.BlockSpec((BLK, D), lambda i: (i, 0)),
            pl.BlockSpec((BLK, D), lambda i: (i, 0)),
            pl.BlockSpec((BLK, D), lambda i: (i, 0)),
            pl.BlockSpec((1, D), lambda i: (0, 0)),
            pl.BlockSpec((D, D_HID), lambda i: (0, 0)),
            pl.BlockSpec((1, D_HID), lambda i: (0, 0)),
            pl.BlockSpec((D_HID, D), lambda i: (0, 0)),
            pl.BlockSpec((1, D), lambda i: (0, 0)),
        ],
        out_specs=pl.BlockSpec((BLK, D), lambda i: (i, 0)),
        out_shape=jax.ShapeDtypeStruct((N, D), jnp.float32),
    )(x, a0, a1, epsv, W1f, b1f, W2f, b2f)


def kernel(input_feature, edge_index, edge_attr, W_e, b_e, eps, W1, b1,
           gamma1, beta1, mean1, var1, W2, b2, gamma2, beta2, mean2, var2):
    src = edge_index[0]
    dst = edge_index[1]

    emb = _edge_encoder(edge_attr, W_e, b_e)
    partials = _sc_aggregate(input_feature, src, dst, emb)
    partials = partials.reshape(NC, NP, D)
    a0 = partials[0]
    a1 = partials[1]

    scale1 = gamma1 / jnp.sqrt(var1 + 1e-5)
    W1f = W1 * scale1[None, :]
    b1f = ((b1 - mean1) * scale1 + beta1).reshape(1, D_HID)
    scale2 = gamma2 / jnp.sqrt(var2 + 1e-5)
    W2f = W2 * scale2[None, :]
    b2f = ((b2 - mean2) * scale2 + beta2).reshape(1, D)
    epsv = jnp.full((1, D), 1.0 + eps, dtype=jnp.float32)

    return _mlp(input_feature, a0, a1, epsv, W1f, b1f, W2f, b2f)
